# baseline (speedup 1.0000x reference)
import functools

import jax
import jax.numpy as jnp
from jax import lax
from jax.experimental import pallas as pl
from jax.experimental.pallas import tpu as pltpu
from jax.experimental.pallas import tpu_sc as plsc

D_MODEL = 64
BATCH = 4096
SEQ = 200
NUM_WORKERS = 32
BLK = 128
PAIR = 2 * D_MODEL
SCALE = 8.0
L = 16
NBUF = 2
PADW = BLK + 1


@functools.partial(
    pl.kernel,
    mesh=plsc.VectorSubcoreMesh(core_axis_name="c", subcore_axis_name="s"),
    out_type=jax.ShapeDtypeStruct((SEQ, D_MODEL, BATCH), jnp.float32),
    compiler_params=pltpu.CompilerParams(
        use_tc_tiling_on_sc=True, needs_layout_passes=False),
    scratch_types=[
        pltpu.VMEM((SEQ, BLK), jnp.int32),
        pltpu.VMEM((NBUF, BLK), jnp.int32),
        pltpu.VMEM((NBUF, BLK, PAIR), jnp.float32),
        pltpu.VMEM((NBUF, D_MODEL, PADW), jnp.float32),
        pltpu.SemaphoreType.DMA((NBUF,)),
        pltpu.SemaphoreType.DMA((NBUF,)),
    ],
)
def _emb_lookup(xt_hbm, t2_hbm, out_hbm, idx_v, idx2_v, pair_v, out_v,
                gsem, ssem):
    w = lax.axis_index("s") * 2 + lax.axis_index("c")
    b0 = w * BLK
    pltpu.sync_copy(xt_hbm.at[:, pl.ds(b0, BLK)], idx_v)

    def prep_and_gather(s, b):
        for l in range(BLK // L):
            idx2_v[b, pl.ds(l * L, L)] = lax.shift_right_logical(
                idx_v[s, pl.ds(l * L, L)], 1)
        pltpu.async_copy(t2_hbm.at[idx2_v.at[b]], pair_v.at[b], gsem.at[b])

    def wait_gather(b):
        pltpu.make_async_copy(t2_hbm.at[idx2_v.at[b]], pair_v.at[b],
                              gsem.at[b]).wait()

    def start_store(s, b):
        pltpu.async_copy(out_v.at[b, :, pl.ds(0, BLK)],
                         out_hbm.at[s, :, pl.ds(b0, BLK)],
                         ssem.at[b])

    def wait_store(s, b):
        pltpu.make_async_copy(out_v.at[b, :, pl.ds(0, BLK)],
                              out_hbm.at[s, :, pl.ds(b0, BLK)],
                              ssem.at[b]).wait()

    for b in range(NBUF):
        prep_and_gather(b, b)

    @pl.loop(0, SEQ, step=NBUF)
    def _outer(s0):
        for b in range(NBUF):
            s = s0 + b
            wait_gather(b)

            @pl.when(s >= NBUF)
            def _():
                wait_store(s - NBUF, b)

            iota = lax.iota(jnp.int32, L)

            @plsc.parallel_loop(0, BLK // L, unroll=2)
            def _shuffle(l):
                parv = lax.shift_left(
                    lax.bitwise_and(idx_v[s, pl.ds(l * L, L)], 1), 6)
                for k in range(L):
                    par = parv[k]
                    bb = l * L + k
                    bvec = jnp.full((L,), 0, jnp.int32) + bb
                    for j in range(D_MODEL // L):
                        val = pair_v[b, bb, pl.ds(par + j * L, L)]
                        plsc.store_scatter(out_v.at[b],
                                           [iota + j * L, bvec],
                                           val * SCALE)

            @pl.when(s + NBUF < SEQ)
            def _():
                prep_and_gather(s + NBUF, b)

            start_store(s, b)

    for b in range(NBUF):
        wait_store(SEQ - NBUF + b, b)


def kernel(x, table):
    t2 = table.reshape(500000, 128)
    xt = x.astype(jnp.int32).T
    out3 = _emb_lookup(xt, t2)
    return jnp.transpose(out3, (2, 0, 1))
